# Initial kernel scaffold; baseline (speedup 1.0000x reference)
import jax
import jax.numpy as jnp
from jax.experimental import pallas as pl


def kernel(input, weight):
    raise NotImplementedError("write your pallas kernel here")


# speedup vs baseline: 1.6846x; 1.6846x over previous
import functools

import jax
import jax.numpy as jnp
from jax import lax
from jax.experimental import pallas as pl
from jax.experimental.pallas import tpu as pltpu
from jax.experimental.pallas import tpu_sc as plsc

EMB_DIM = 64
BATCH, SEQ = 16384, 50
B_TOTAL = BATCH * SEQ
NUM_CORES = 2
NUM_SUBCORES = 16
NW = NUM_CORES * NUM_SUBCORES
B_PER_W = B_TOTAL // NW
CHUNK = 128
N_CHUNKS = B_PER_W // CHUNK


def _make_gather():
    mesh = plsc.VectorSubcoreMesh(core_axis_name="c", subcore_axis_name="s")

    @functools.partial(
        pl.kernel,
        mesh=mesh,
        out_type=jax.ShapeDtypeStruct((B_TOTAL, EMB_DIM), jnp.float32),
        scratch_types=[
            pltpu.VMEM((N_CHUNKS, CHUNK), jnp.int32),
            pltpu.VMEM((CHUNK, EMB_DIM), jnp.float32),
            pltpu.SemaphoreType.DMA,
        ],
        compiler_params=pltpu.CompilerParams(use_tc_tiling_on_sc=False),
    )
    def gather_kernel(idx_hbm, table_hbm, out_hbm, idx_v, rows_v, sem):
        wid = lax.axis_index("s") * NUM_CORES + lax.axis_index("c")
        base = wid * B_PER_W
        pltpu.sync_copy(idx_hbm.at[wid], idx_v)

        def body(j, carry):
            pltpu.async_copy(table_hbm.at[idx_v.at[j]], rows_v, sem).wait()
            pltpu.sync_copy(rows_v, out_hbm.at[pl.ds(base + j * CHUNK, CHUNK)])
            return carry

        lax.fori_loop(0, N_CHUNKS, body, 0, unroll=False)

    return gather_kernel


_gather = _make_gather()


def kernel(input, weight):
    idx = input.reshape(NW, N_CHUNKS, CHUNK)
    out = _gather(idx, weight)
    return out.reshape(BATCH, SEQ, EMB_DIM)


# speedup vs baseline: 1.8769x; 1.1142x over previous
import functools

import jax
import jax.numpy as jnp
from jax import lax
from jax.experimental import pallas as pl
from jax.experimental.pallas import tpu as pltpu
from jax.experimental.pallas import tpu_sc as plsc

EMB_DIM = 64
BATCH, SEQ = 16384, 50
B_TOTAL = BATCH * SEQ
NUM_CORES = 2
NUM_SUBCORES = 16
NW = NUM_CORES * NUM_SUBCORES
B_PER_W = B_TOTAL // NW
CHUNK = 128
N_CHUNKS = B_PER_W // CHUNK
K = 2
GROUP = K * CHUNK
G = B_PER_W // GROUP
NBUF = 4


def _make_gather():
    mesh = plsc.VectorSubcoreMesh(core_axis_name="c", subcore_axis_name="s")

    @functools.partial(
        pl.kernel,
        mesh=mesh,
        out_type=jax.ShapeDtypeStruct((B_TOTAL, EMB_DIM), jnp.float32),
        scratch_types=[
            pltpu.VMEM((N_CHUNKS, CHUNK), jnp.int32),
            pltpu.VMEM((NBUF, GROUP, EMB_DIM), jnp.float32),
            pltpu.SemaphoreType.DMA((NBUF,)),
            pltpu.SemaphoreType.DMA((NBUF,)),
        ],
        compiler_params=pltpu.CompilerParams(use_tc_tiling_on_sc=False),
    )
    def gather_kernel(idx_hbm, table_hbm, out_hbm, idx_v, rows_v, gsem, wsem):
        wid = lax.axis_index("s") * NUM_CORES + lax.axis_index("c")
        base = wid * B_PER_W
        pltpu.sync_copy(idx_hbm.at[wid], idx_v)

        def fire_group(grp, buf):
            for b in range(K):
                pltpu.async_copy(
                    table_hbm.at[idx_v.at[grp * K + b]],
                    rows_v.at[buf, pl.ds(b * CHUNK, CHUNK)],
                    gsem.at[buf],
                )

        def wait_group(buf):
            for b in range(K):
                pltpu.make_async_copy(
                    table_hbm.at[idx_v.at[0]],
                    rows_v.at[buf, pl.ds(b * CHUNK, CHUNK)],
                    gsem.at[buf],
                ).wait()

        def fire_write(grp, buf):
            pltpu.async_copy(
                rows_v.at[buf],
                out_hbm.at[pl.ds(base + grp * GROUP, GROUP)],
                wsem.at[buf],
            )

        def wait_write(buf):
            pltpu.make_async_copy(
                rows_v.at[buf],
                out_hbm.at[pl.ds(base, GROUP)],
                wsem.at[buf],
            ).wait()

        for g in range(NBUF - 1):
            fire_group(g, g)

        def body(i, carry):
            for b in range(NBUF):
                grp = i * NBUF + b
                ahead = (b + NBUF - 1) % NBUF

                @pl.when(grp + NBUF - 1 <= G - 1)
                def _fire():
                    @pl.when(grp >= 1)
                    def _drain():
                        wait_write(ahead)

                    fire_group(grp + NBUF - 1, ahead)

                wait_group(b)
                fire_write(grp, b)
            return carry

        lax.fori_loop(0, G // NBUF, body, 0, unroll=False)

        for b in range(NBUF):
            wait_write(b)

    return gather_kernel


_gather = _make_gather()


def kernel(input, weight):
    idx = input.reshape(NW, N_CHUNKS, CHUNK)
    out = _gather(idx, weight)
    return out.reshape(BATCH, SEQ, EMB_DIM)
